# baseline (speedup 1.0000x reference)
import functools
import math

import numpy as np
import jax
import jax.numpy as jnp
from jax import lax
from jax.experimental import pallas as pl
from jax.experimental.pallas import tpu as pltpu
from jax.experimental.pallas import tpu_sc as plsc

SS = ((64, 64), (32, 32), (16, 16), (8, 8))
BS, NQ, C, NH, NL, NP = 4, 900, 256, 8, 4, 4
HD = C // NH
LP = NL * NP
NV = sum(h * w for h, w in SS)
NR = BS * NQ
NROWS = BS * NV * NH

RB = 16
NBLK = NR // RB
NTILES = 32
SC_ITERS = -(-NBLK // NTILES)


def _matmul_body(x_ref, w_ref, b_ref, o_ref):
    o_ref[:] = jnp.dot(x_ref[:], w_ref[:],
                       preferred_element_type=jnp.float32,
                       precision=lax.Precision.HIGHEST) + b_ref[:]


def _matmul(x, w, b, bm):
    m, k = x.shape
    n = w.shape[1]
    return pl.pallas_call(
        _matmul_body,
        grid=(m // bm,),
        in_specs=[pl.BlockSpec((bm, k), lambda i: (i, 0)),
                  pl.BlockSpec((k, n), lambda i: (0, 0)),
                  pl.BlockSpec((1, n), lambda i: (0, 0))],
        out_specs=pl.BlockSpec((bm, n), lambda i: (i, 0)),
        out_shape=jax.ShapeDtypeStruct((m, n), jnp.float32),
    )(x, w, b.reshape(1, n))


QBM = 400


def _qside_body(q_ref, rp_ref, wq_ref, bq_ref, sel_ref, bd_ref, cst_ref,
                idx_ref, w_ref):
    pid = pl.program_id(0)
    t = jnp.dot(q_ref[:], wq_ref[:],
                preferred_element_type=jnp.float32,
                precision=lax.Precision.HIGHEST) + bq_ref[:]
    offx = t[:, 0:128]
    offy = t[:, 128:256]
    logits = t[:, 256:384]

    m = jnp.max(logits, axis=1, keepdims=True)
    e = jnp.exp(logits - m)
    denom = jnp.dot(e, bd_ref[:], preferred_element_type=jnp.float32,
                    precision=lax.Precision.HIGHEST)
    aw = e / denom

    rxy = jnp.dot(rp_ref[:], sel_ref[:], preferred_element_type=jnp.float32,
                  precision=lax.Precision.HIGHEST)
    refx = rxy[:, 0:128]
    refy = rxy[:, 128:256]

    wl = cst_ref[0:1, :]
    hl = cst_ref[1:2, :]
    bl = cst_ref[2:3, :]
    hlane = cst_ref[3:4, :]

    ix = refx * wl + offx - 0.5
    iy = refy * hl + offy - 0.5
    big = 16777216.0
    ix = jnp.clip(ix, -4.0, big)
    iy = jnp.clip(iy, -4.0, big)
    x0 = jnp.floor(ix)
    fx = ix - x0
    y0 = jnp.floor(iy)
    fy = iy - y0

    rows = lax.broadcasted_iota(jnp.int32, (QBM, 1), 0) + pid * QBM
    bvec = (rows // NQ).astype(jnp.float32)
    base_all = bvec * float(NV) + bl

    for c, (cy, cx) in enumerate(((0, 0), (0, 1), (1, 0), (1, 1))):
        xq = x0 + float(cx)
        yq = y0 + float(cy)
        valid = ((xq >= 0.0) & (xq <= wl - 1.0)
                 & (yq >= 0.0) & (yq <= hl - 1.0)).astype(jnp.float32)
        xc = jnp.clip(xq, 0.0, wl - 1.0)
        yc = jnp.clip(yq, 0.0, hl - 1.0)
        flat = (base_all + yc * wl + xc) * float(NH) + hlane
        wx = (1.0 - fx) if cx == 0 else fx
        wy = (1.0 - fy) if cy == 0 else fy
        idx_ref[c, :, :] = flat.astype(jnp.int32)
        w_ref[c, :, :] = aw * wx * wy * valid


def _qside(q2, rp2, wq, bq, sel, bd, cst):
    return pl.pallas_call(
        _qside_body,
        grid=(NR // QBM,),
        in_specs=[pl.BlockSpec((QBM, C), lambda i: (i, 0)),
                  pl.BlockSpec((QBM, 2 * NL), lambda i: (i, 0)),
                  pl.BlockSpec((C, 384), lambda i: (0, 0)),
                  pl.BlockSpec((1, 384), lambda i: (0, 0)),
                  pl.BlockSpec((2 * NL, 256), lambda i: (0, 0)),
                  pl.BlockSpec((128, 128), lambda i: (0, 0)),
                  pl.BlockSpec((8, 128), lambda i: (0, 0))],
        out_specs=[pl.BlockSpec((4, QBM, 128), lambda i: (0, i, 0)),
                   pl.BlockSpec((4, QBM, 128), lambda i: (0, i, 0))],
        out_shape=[jax.ShapeDtypeStruct((4, NR, 128), jnp.int32),
                   jax.ShapeDtypeStruct((4, NR, 128), jnp.float32)],
    )(q2, rp2, wq, bq, sel, bd, cst)


def _sc_body(idx_hbm, w_hbm, vrows_hbm, out_hbm,
             idxv, wv, rows, outb, sem0, sem1):
    wid = lax.axis_index("s") * 2 + lax.axis_index("c")

    def fire(j, slot, sem):
        for c in range(4):
            pltpu.async_copy(vrows_hbm.at[idxv.at[c, j]],
                             rows.at[slot, pl.ds(c * 128, 128)], sem)

    def drain(j, slot, sem):
        for c in range(4):
            pltpu.make_async_copy(vrows_hbm.at[idxv.at[c, j]],
                                  rows.at[slot, pl.ds(c * 128, 128)],
                                  sem).wait()

    def compute(j, slot):
        def h_body(h, carry):
            hbase = h * 16
            parts0 = []
            parts1 = []
            for c in range(4):
                w16 = wv[c, j, pl.ds(hbase, 16)]
                a0e = jnp.zeros((16,), jnp.float32)
                a0o = jnp.zeros((16,), jnp.float32)
                a1e = jnp.zeros((16,), jnp.float32)
                a1o = jnp.zeros((16,), jnp.float32)
                for lp in range(16):
                    wb = jnp.full((16,), w16[lp], jnp.float32)
                    ri = hbase + (c * 128 + lp)
                    r0 = rows[slot, ri, pl.ds(0, 16)]
                    r1 = rows[slot, ri, pl.ds(16, 16)]
                    if lp % 2 == 0:
                        a0e = a0e + wb * r0
                        a1e = a1e + wb * r1
                    else:
                        a0o = a0o + wb * r0
                        a1o = a1o + wb * r1
                parts0.append(a0e + a0o)
                parts1.append(a1e + a1o)
            acc0 = (parts0[0] + parts0[1]) + (parts0[2] + parts0[3])
            acc1 = (parts1[0] + parts1[1]) + (parts1[2] + parts1[3])
            outb[j * 8 + h, pl.ds(0, 16)] = acc0
            outb[j * 8 + h, pl.ds(16, 16)] = acc1
            return carry
        lax.fori_loop(0, NH, h_body, 0)

    def blk_body(i, carry):
        bid = i * NTILES + wid

        @pl.when(bid < NBLK)
        def _():
            r0 = bid * RB
            pltpu.sync_copy(idx_hbm.at[:, pl.ds(r0, RB)], idxv)
            pltpu.sync_copy(w_hbm.at[:, pl.ds(r0, RB)], wv)
            fire(0, 0, sem0)

            def pair_body(p, c2):
                j0 = 2 * p
                fire(j0 + 1, 1, sem1)
                drain(j0, 0, sem0)
                compute(j0, 0)

                @pl.when(p < RB // 2 - 1)
                def _():
                    fire(j0 + 2, 0, sem0)
                drain(j0 + 1, 1, sem1)
                compute(j0 + 1, 1)
                return c2
            lax.fori_loop(0, RB // 2, pair_body, 0)
            pltpu.sync_copy(outb, out_hbm.at[pl.ds(r0 * NH, RB * NH)])
        return carry

    lax.fori_loop(0, SC_ITERS, blk_body, 0)


@functools.cache
def _sc_gather_fn():
    mesh = plsc.VectorSubcoreMesh(core_axis_name="c", subcore_axis_name="s",
                                  num_cores=2, num_subcores=16)
    return pl.kernel(
        _sc_body,
        out_type=jax.ShapeDtypeStruct((NR * NH, HD), jnp.float32),
        mesh=mesh,
        scratch_types=[
            pltpu.VMEM((4, RB, 128), jnp.int32),
            pltpu.VMEM((4, RB, 128), jnp.float32),
            pltpu.VMEM((2, 512, HD), jnp.float32),
            pltpu.VMEM((RB * NH, HD), jnp.float32),
            pltpu.SemaphoreType.DMA,
            pltpu.SemaphoreType.DMA,
        ],
        compiler_params=pltpu.CompilerParams(use_tc_tiling_on_sc=False),
    )


def _host_constants():
    hlp = np.arange(NH * NL * NP)
    perm = np.concatenate([hlp * 2 + 0, hlp * 2 + 1]).astype(np.int32)
    l_of_lane = (hlp % LP) // NP
    h_of_lane = hlp // LP
    wl = np.array([SS[l][1] for l in l_of_lane], np.float32)
    hl = np.array([SS[l][0] for l in l_of_lane], np.float32)
    bases = np.cumsum([0] + [h * w for h, w in SS])[:-1]
    bl = np.array([bases[l] for l in l_of_lane], np.float32)
    cst = np.zeros((8, 128), np.float32)
    cst[0] = wl
    cst[1] = hl
    cst[2] = bl
    cst[3] = h_of_lane.astype(np.float32)
    sel = np.zeros((2 * NL, 256), np.float32)
    for lane in range(128):
        sel[l_of_lane[lane] * 2 + 0, lane] = 1.0
        sel[l_of_lane[lane] * 2 + 1, 128 + lane] = 1.0
    bd = np.kron(np.eye(NH, dtype=np.float32),
                 np.ones((LP, LP), np.float32))
    return perm, cst, sel, bd


_PERM, _CST, _SEL, _BD = _host_constants()


def kernel(query, reference_points, value, value_spatial_shapes,
           W_off, b_off, W_attn, b_attn, W_val, b_val, W_out, b_out):
    wq = jnp.concatenate([W_off[_PERM], W_attn], axis=0).T
    bq = jnp.concatenate([b_off[_PERM], b_attn], axis=0).reshape(1, 384)

    vrows = _matmul(value.reshape(BS * NV, C), W_val.T, b_val, 1360)
    vrows = vrows.reshape(NROWS, HD)

    idx, w = _qside(query.reshape(NR, C),
                    reference_points.reshape(NR, 2 * NL),
                    wq, bq,
                    jnp.asarray(_SEL), jnp.asarray(_BD), jnp.asarray(_CST))

    s = _sc_gather_fn()(idx, w, vrows)

    out = _matmul(s.reshape(NR, C), W_out.T, b_out, 720)
    return out.reshape(BS, NQ, C)


# speedup vs baseline: 1.2700x; 1.2700x over previous
import functools
import math

import numpy as np
import jax
import jax.numpy as jnp
from jax import lax
from jax.experimental import pallas as pl
from jax.experimental.pallas import tpu as pltpu
from jax.experimental.pallas import tpu_sc as plsc

SS = ((64, 64), (32, 32), (16, 16), (8, 8))
BS, NQ, C, NH, NL, NP = 4, 900, 256, 8, 4, 4
HD = C // NH
LP = NL * NP
NV = sum(h * w for h, w in SS)
NR = BS * NQ
NROWS = BS * NV * NH

RB = 16
NBLK = NR // RB
NTILES = 32
SC_ITERS = -(-NBLK // NTILES)


def _matmul_body(x_ref, w_ref, b_ref, o_ref):
    r = jnp.dot(x_ref[:], w_ref[:],
                preferred_element_type=jnp.float32,
                precision=lax.Precision.HIGHEST) + b_ref[:]
    o_ref[:] = r.astype(o_ref.dtype)


def _matmul(x, w, b, bm, out_dtype=jnp.float32):
    m, k = x.shape
    n = w.shape[1]
    return pl.pallas_call(
        _matmul_body,
        grid=(m // bm,),
        in_specs=[pl.BlockSpec((bm, k), lambda i: (i, 0)),
                  pl.BlockSpec((k, n), lambda i: (0, 0)),
                  pl.BlockSpec((1, n), lambda i: (0, 0))],
        out_specs=pl.BlockSpec((bm, n), lambda i: (i, 0)),
        out_shape=jax.ShapeDtypeStruct((m, n), out_dtype),
    )(x, w, b.reshape(1, n))


QBM = 400


def _qside_body(q_ref, rp_ref, wq_ref, bq_ref, sel_ref, bd_ref, cst_ref,
                idx_ref, w_ref):
    pid = pl.program_id(0)
    t = jnp.dot(q_ref[:], wq_ref[:],
                preferred_element_type=jnp.float32,
                precision=lax.Precision.HIGHEST) + bq_ref[:]
    offx = t[:, 0:128]
    offy = t[:, 128:256]
    logits = t[:, 256:384]

    m = jnp.max(logits, axis=1, keepdims=True)
    e = jnp.exp(logits - m)
    denom = jnp.dot(e, bd_ref[:], preferred_element_type=jnp.float32,
                    precision=lax.Precision.HIGHEST)
    aw = e / denom

    rxy = jnp.dot(rp_ref[:], sel_ref[:], preferred_element_type=jnp.float32,
                  precision=lax.Precision.HIGHEST)
    refx = rxy[:, 0:128]
    refy = rxy[:, 128:256]

    wl = cst_ref[0:1, :]
    hl = cst_ref[1:2, :]
    bl = cst_ref[2:3, :]
    hlane = cst_ref[3:4, :]

    ix = refx * wl + offx - 0.5
    iy = refy * hl + offy - 0.5
    big = 16777216.0
    ix = jnp.clip(ix, -4.0, big)
    iy = jnp.clip(iy, -4.0, big)
    x0 = jnp.floor(ix)
    fx = ix - x0
    y0 = jnp.floor(iy)
    fy = iy - y0

    rows = lax.broadcasted_iota(jnp.int32, (QBM, 1), 0) + pid * QBM
    bvec = (rows // NQ).astype(jnp.float32)
    base_all = bvec * float(NV) + bl

    for c, (cy, cx) in enumerate(((0, 0), (0, 1), (1, 0), (1, 1))):
        xq = x0 + float(cx)
        yq = y0 + float(cy)
        valid = ((xq >= 0.0) & (xq <= wl - 1.0)
                 & (yq >= 0.0) & (yq <= hl - 1.0)).astype(jnp.float32)
        xc = jnp.clip(xq, 0.0, wl - 1.0)
        yc = jnp.clip(yq, 0.0, hl - 1.0)
        flat = (base_all + yc * wl + xc) * float(NH) + hlane
        wx = (1.0 - fx) if cx == 0 else fx
        wy = (1.0 - fy) if cy == 0 else fy
        idx_ref[c, :, :] = flat.astype(jnp.int32)
        w_ref[c, :, :] = aw * wx * wy * valid


def _qside(q2, rp2, wq, bq, sel, bd, cst):
    return pl.pallas_call(
        _qside_body,
        grid=(NR // QBM,),
        in_specs=[pl.BlockSpec((QBM, C), lambda i: (i, 0)),
                  pl.BlockSpec((QBM, 2 * NL), lambda i: (i, 0)),
                  pl.BlockSpec((C, 384), lambda i: (0, 0)),
                  pl.BlockSpec((1, 384), lambda i: (0, 0)),
                  pl.BlockSpec((2 * NL, 256), lambda i: (0, 0)),
                  pl.BlockSpec((128, 128), lambda i: (0, 0)),
                  pl.BlockSpec((8, 128), lambda i: (0, 0))],
        out_specs=[pl.BlockSpec((4, QBM, 128), lambda i: (0, i, 0)),
                   pl.BlockSpec((4, QBM, 128), lambda i: (0, i, 0))],
        out_shape=[jax.ShapeDtypeStruct((4, NR, 128), jnp.int32),
                   jax.ShapeDtypeStruct((4, NR, 128), jnp.float32)],
    )(q2, rp2, wq, bq, sel, bd, cst)


def _sc_body(idx_hbm, w_hbm, vrows_hbm, out_hbm,
             idxv, wv, rows, outb, sem0, sem1):
    wid = lax.axis_index("s") * 2 + lax.axis_index("c")

    def fire(j, slot, sem):
        for c in range(4):
            pltpu.async_copy(vrows_hbm.at[idxv.at[c, j]],
                             rows.at[slot, pl.ds(c * 128, 128)], sem)

    def drain(j, slot, sem):
        for c in range(4):
            pltpu.make_async_copy(vrows_hbm.at[idxv.at[c, j]],
                                  rows.at[slot, pl.ds(c * 128, 128)],
                                  sem).wait()

    def compute(j, slot):
        def h_body(h, carry):
            hbase = h * 16
            parts0 = []
            parts1 = []
            hi_mask = jnp.full((16,), 0xFFFF0000, jnp.uint32)
            for c in range(4):
                w16 = wv[c, j, pl.ds(hbase, 16)]
                a0e = jnp.zeros((16,), jnp.float32)
                a0o = jnp.zeros((16,), jnp.float32)
                a1e = jnp.zeros((16,), jnp.float32)
                a1o = jnp.zeros((16,), jnp.float32)
                for lp in range(16):
                    wb = jnp.full((16,), w16[lp], jnp.float32)
                    ri = hbase + (c * 128 + lp)
                    u = plsc.bitcast(rows[slot, ri, :], jnp.uint32)
                    r0 = plsc.bitcast(u << 16, jnp.float32)
                    r1 = plsc.bitcast(u & hi_mask, jnp.float32)
                    if lp % 2 == 0:
                        a0e = a0e + wb * r0
                        a1e = a1e + wb * r1
                    else:
                        a0o = a0o + wb * r0
                        a1o = a1o + wb * r1
                parts0.append(a0e + a0o)
                parts1.append(a1e + a1o)
            acc0 = (parts0[0] + parts0[1]) + (parts0[2] + parts0[3])
            acc1 = (parts1[0] + parts1[1]) + (parts1[2] + parts1[3])
            outb[j * 8 + h, pl.ds(0, 16)] = acc0
            outb[j * 8 + h, pl.ds(16, 16)] = acc1
            return carry
        lax.fori_loop(0, NH, h_body, 0)

    def blk_body(i, carry):
        bid = i * NTILES + wid

        @pl.when(bid < NBLK)
        def _():
            r0 = bid * RB
            pltpu.sync_copy(idx_hbm.at[:, pl.ds(r0, RB)], idxv)
            pltpu.sync_copy(w_hbm.at[:, pl.ds(r0, RB)], wv)
            fire(0, 0, sem0)

            def pair_body(p, c2):
                j0 = 2 * p
                fire(j0 + 1, 1, sem1)
                drain(j0, 0, sem0)
                compute(j0, 0)

                @pl.when(p < RB // 2 - 1)
                def _():
                    fire(j0 + 2, 0, sem0)
                drain(j0 + 1, 1, sem1)
                compute(j0 + 1, 1)
                return c2
            lax.fori_loop(0, RB // 2, pair_body, 0)
            pltpu.sync_copy(outb, out_hbm.at[pl.ds(r0 * NH, RB * NH)])
        return carry

    lax.fori_loop(0, SC_ITERS, blk_body, 0)


@functools.cache
def _sc_gather_fn():
    mesh = plsc.VectorSubcoreMesh(core_axis_name="c", subcore_axis_name="s",
                                  num_cores=2, num_subcores=16)
    return pl.kernel(
        _sc_body,
        out_type=jax.ShapeDtypeStruct((NR * NH, HD), jnp.float32),
        mesh=mesh,
        scratch_types=[
            pltpu.VMEM((4, RB, 128), jnp.int32),
            pltpu.VMEM((4, RB, 128), jnp.float32),
            pltpu.VMEM((2, 512, HD), jnp.bfloat16),
            pltpu.VMEM((RB * NH, HD), jnp.float32),
            pltpu.SemaphoreType.DMA,
            pltpu.SemaphoreType.DMA,
        ],
        compiler_params=pltpu.CompilerParams(use_tc_tiling_on_sc=False,
                                             needs_layout_passes=False),
    )


def _host_constants():
    hlp = np.arange(NH * NL * NP)
    perm = np.concatenate([hlp * 2 + 0, hlp * 2 + 1]).astype(np.int32)
    l_of_lane = (hlp % LP) // NP
    h_of_lane = hlp // LP
    wl = np.array([SS[l][1] for l in l_of_lane], np.float32)
    hl = np.array([SS[l][0] for l in l_of_lane], np.float32)
    bases = np.cumsum([0] + [h * w for h, w in SS])[:-1]
    bl = np.array([bases[l] for l in l_of_lane], np.float32)
    cst = np.zeros((8, 128), np.float32)
    cst[0] = wl
    cst[1] = hl
    cst[2] = bl
    cst[3] = h_of_lane.astype(np.float32)
    sel = np.zeros((2 * NL, 256), np.float32)
    for lane in range(128):
        sel[l_of_lane[lane] * 2 + 0, lane] = 1.0
        sel[l_of_lane[lane] * 2 + 1, 128 + lane] = 1.0
    bd = np.kron(np.eye(NH, dtype=np.float32),
                 np.ones((LP, LP), np.float32))
    cp = np.zeros(32, np.int32)
    cp[0::2] = np.arange(16)
    cp[1::2] = np.arange(16) + 16
    colperm = (np.arange(NH)[:, None] * 32 + cp[None, :]).reshape(-1)
    return perm, cst, sel, bd, colperm


_PERM, _CST, _SEL, _BD, _COLPERM = _host_constants()


def kernel(query, reference_points, value, value_spatial_shapes,
           W_off, b_off, W_attn, b_attn, W_val, b_val, W_out, b_out):
    wq = jnp.concatenate([W_off[_PERM], W_attn], axis=0).T
    bq = jnp.concatenate([b_off[_PERM], b_attn], axis=0).reshape(1, 384)

    vrows = _matmul(value.reshape(BS * NV, C), W_val.T[:, _COLPERM],
                    b_val[_COLPERM], 1360, out_dtype=jnp.bfloat16)
    vrows = vrows.reshape(NROWS, HD)

    idx, w = _qside(query.reshape(NR, C),
                    reference_points.reshape(NR, 2 * NL),
                    wq, bq,
                    jnp.asarray(_SEL), jnp.asarray(_BD), jnp.asarray(_CST))

    s = _sc_gather_fn()(idx, w, vrows)

    out = _matmul(s.reshape(NR, C), W_out.T, b_out, 720)
    return out.reshape(BS, NQ, C)


# speedup vs baseline: 1.3205x; 1.0398x over previous
import functools
import math

import numpy as np
import jax
import jax.numpy as jnp
from jax import lax
from jax.experimental import pallas as pl
from jax.experimental.pallas import tpu as pltpu
from jax.experimental.pallas import tpu_sc as plsc

SS = ((64, 64), (32, 32), (16, 16), (8, 8))
BS, NQ, C, NH, NL, NP = 4, 900, 256, 8, 4, 4
HD = C // NH
LP = NL * NP
NV = sum(h * w for h, w in SS)
NR = BS * NQ
NROWS = BS * NV * NH

RB = 16
NBLK = NR // RB
NTILES = 32
SC_ITERS = -(-NBLK // NTILES)


def _matmul_body(x_ref, w_ref, b_ref, o_ref):
    r = jnp.dot(x_ref[:], w_ref[:],
                preferred_element_type=jnp.float32,
                precision=lax.Precision.HIGHEST) + b_ref[:]
    o_ref[:] = r.astype(o_ref.dtype)


def _matmul(x, w, b, bm, out_dtype=jnp.float32):
    m, k = x.shape
    n = w.shape[1]
    return pl.pallas_call(
        _matmul_body,
        grid=(m // bm,),
        in_specs=[pl.BlockSpec((bm, k), lambda i: (i, 0)),
                  pl.BlockSpec((k, n), lambda i: (0, 0)),
                  pl.BlockSpec((1, n), lambda i: (0, 0))],
        out_specs=pl.BlockSpec((bm, n), lambda i: (i, 0)),
        out_shape=jax.ShapeDtypeStruct((m, n), out_dtype),
    )(x, w, b.reshape(1, n))


QBM = 400


def _qside_body(q_ref, rp_ref, wq_ref, bq_ref, sel_ref, bd_ref, cst_ref,
                idx_ref, w_ref):
    pid = pl.program_id(0)
    t = jnp.dot(q_ref[:], wq_ref[:],
                preferred_element_type=jnp.float32,
                precision=lax.Precision.HIGHEST) + bq_ref[:]
    offx = t[:, 0:128]
    offy = t[:, 128:256]
    logits = t[:, 256:384]

    m = jnp.max(logits, axis=1, keepdims=True)
    e = jnp.exp(logits - m)
    denom = jnp.dot(e, bd_ref[:], preferred_element_type=jnp.float32,
                    precision=lax.Precision.HIGHEST)
    aw = e / denom

    rxy = jnp.dot(rp_ref[:], sel_ref[:], preferred_element_type=jnp.float32,
                  precision=lax.Precision.HIGHEST)
    refx = rxy[:, 0:128]
    refy = rxy[:, 128:256]

    wl = cst_ref[0:1, :]
    hl = cst_ref[1:2, :]
    bl = cst_ref[2:3, :]
    hlane = cst_ref[3:4, :]

    ix = refx * wl + offx - 0.5
    iy = refy * hl + offy - 0.5
    big = 16777216.0
    ix = jnp.clip(ix, -4.0, big)
    iy = jnp.clip(iy, -4.0, big)
    x0 = jnp.floor(ix)
    fx = ix - x0
    y0 = jnp.floor(iy)
    fy = iy - y0

    rows = lax.broadcasted_iota(jnp.int32, (QBM, 1), 0) + pid * QBM
    bvec = (rows // NQ).astype(jnp.float32)
    base_all = bvec * float(NV) + bl

    for c, (cy, cx) in enumerate(((0, 0), (0, 1), (1, 0), (1, 1))):
        xq = x0 + float(cx)
        yq = y0 + float(cy)
        valid = ((xq >= 0.0) & (xq <= wl - 1.0)
                 & (yq >= 0.0) & (yq <= hl - 1.0)).astype(jnp.float32)
        xc = jnp.clip(xq, 0.0, wl - 1.0)
        yc = jnp.clip(yq, 0.0, hl - 1.0)
        flat = (base_all + yc * wl + xc) * float(NH) + hlane
        wx = (1.0 - fx) if cx == 0 else fx
        wy = (1.0 - fy) if cy == 0 else fy
        idx_ref[c, :, :] = flat.astype(jnp.int32)
        w_ref[c, :, :] = aw * wx * wy * valid


def _qside(q2, rp2, wq, bq, sel, bd, cst):
    return pl.pallas_call(
        _qside_body,
        grid=(NR // QBM,),
        in_specs=[pl.BlockSpec((QBM, C), lambda i: (i, 0)),
                  pl.BlockSpec((QBM, 2 * NL), lambda i: (i, 0)),
                  pl.BlockSpec((C, 384), lambda i: (0, 0)),
                  pl.BlockSpec((1, 384), lambda i: (0, 0)),
                  pl.BlockSpec((2 * NL, 256), lambda i: (0, 0)),
                  pl.BlockSpec((128, 128), lambda i: (0, 0)),
                  pl.BlockSpec((8, 128), lambda i: (0, 0))],
        out_specs=[pl.BlockSpec((4, QBM, 128), lambda i: (0, i, 0)),
                   pl.BlockSpec((4, QBM, 128), lambda i: (0, i, 0))],
        out_shape=[jax.ShapeDtypeStruct((4, NR, 128), jnp.int32),
                   jax.ShapeDtypeStruct((4, NR, 128), jnp.float32)],
    )(q2, rp2, wq, bq, sel, bd, cst)


def _sc_body(idx_hbm, w_hbm, vrows_hbm, out_hbm,
             idxv, wv, rows, outb, sem0, sem1, sem2, sem3):
    wid = lax.axis_index("s") * 2 + lax.axis_index("c")
    sems = (sem0, sem1, sem2, sem3)

    def fire(j, slot, sem):
        for c in range(4):
            pltpu.async_copy(vrows_hbm.at[idxv.at[c, j]],
                             rows.at[slot, pl.ds(c * 128, 128)], sem)

    def drain(j, slot, sem):
        for c in range(4):
            pltpu.make_async_copy(vrows_hbm.at[idxv.at[c, j]],
                                  rows.at[slot, pl.ds(c * 128, 128)],
                                  sem).wait()

    def compute(j, slot):
        def h_body(h, carry):
            hbase = h * 16
            parts0 = []
            parts1 = []
            hi_mask = jnp.full((16,), 0xFFFF0000, jnp.uint32)
            for c in range(4):
                w16 = wv[c, j, pl.ds(hbase, 16)]
                a0e = jnp.zeros((16,), jnp.float32)
                a0o = jnp.zeros((16,), jnp.float32)
                a1e = jnp.zeros((16,), jnp.float32)
                a1o = jnp.zeros((16,), jnp.float32)
                for lp in range(16):
                    wb = jnp.full((16,), w16[lp], jnp.float32)
                    ri = hbase + (c * 128 + lp)
                    u = plsc.bitcast(rows[slot, ri, :], jnp.uint32)
                    r0 = plsc.bitcast(u << 16, jnp.float32)
                    r1 = plsc.bitcast(u & hi_mask, jnp.float32)
                    if lp % 2 == 0:
                        a0e = a0e + wb * r0
                        a1e = a1e + wb * r1
                    else:
                        a0o = a0o + wb * r0
                        a1o = a1o + wb * r1
                parts0.append(a0e + a0o)
                parts1.append(a1e + a1o)
            acc0 = (parts0[0] + parts0[1]) + (parts0[2] + parts0[3])
            acc1 = (parts1[0] + parts1[1]) + (parts1[2] + parts1[3])
            outb[j * 8 + h, pl.ds(0, 16)] = acc0
            outb[j * 8 + h, pl.ds(16, 16)] = acc1
            return carry
        lax.fori_loop(0, NH, h_body, 0)

    def blk_body(i, carry):
        bid = i * NTILES + wid

        @pl.when(bid < NBLK)
        def _():
            r0 = bid * RB
            pltpu.sync_copy(idx_hbm.at[:, pl.ds(r0, RB)], idxv)
            pltpu.sync_copy(w_hbm.at[:, pl.ds(r0, RB)], wv)
            fire(0, 0, sems[0])
            fire(1, 1, sems[1])
            fire(2, 2, sems[2])

            def quad_body(p, c2):
                j = 4 * p
                for q in range(4):
                    jq = j + q

                    @pl.when(jq + 3 < RB)
                    def _():
                        fire(jq + 3, (q + 3) % 4, sems[(q + 3) % 4])
                    drain(jq, q, sems[q])
                    compute(jq, q)
                return c2
            lax.fori_loop(0, RB // 4, quad_body, 0)
            pltpu.sync_copy(outb, out_hbm.at[pl.ds(r0 * NH, RB * NH)])
        return carry

    lax.fori_loop(0, SC_ITERS, blk_body, 0)


@functools.cache
def _sc_gather_fn():
    mesh = plsc.VectorSubcoreMesh(core_axis_name="c", subcore_axis_name="s",
                                  num_cores=2, num_subcores=16)
    return pl.kernel(
        _sc_body,
        out_type=jax.ShapeDtypeStruct((NR * NH, HD), jnp.float32),
        mesh=mesh,
        scratch_types=[
            pltpu.VMEM((4, RB, 128), jnp.int32),
            pltpu.VMEM((4, RB, 128), jnp.float32),
            pltpu.VMEM((4, 512, HD), jnp.bfloat16),
            pltpu.VMEM((RB * NH, HD), jnp.float32),
            pltpu.SemaphoreType.DMA,
            pltpu.SemaphoreType.DMA,
            pltpu.SemaphoreType.DMA,
            pltpu.SemaphoreType.DMA,
        ],
        compiler_params=pltpu.CompilerParams(use_tc_tiling_on_sc=False,
                                             needs_layout_passes=False),
    )


def _host_constants():
    hlp = np.arange(NH * NL * NP)
    perm = np.concatenate([hlp * 2 + 0, hlp * 2 + 1]).astype(np.int32)
    l_of_lane = (hlp % LP) // NP
    h_of_lane = hlp // LP
    wl = np.array([SS[l][1] for l in l_of_lane], np.float32)
    hl = np.array([SS[l][0] for l in l_of_lane], np.float32)
    bases = np.cumsum([0] + [h * w for h, w in SS])[:-1]
    bl = np.array([bases[l] for l in l_of_lane], np.float32)
    cst = np.zeros((8, 128), np.float32)
    cst[0] = wl
    cst[1] = hl
    cst[2] = bl
    cst[3] = h_of_lane.astype(np.float32)
    sel = np.zeros((2 * NL, 256), np.float32)
    for lane in range(128):
        sel[l_of_lane[lane] * 2 + 0, lane] = 1.0
        sel[l_of_lane[lane] * 2 + 1, 128 + lane] = 1.0
    bd = np.kron(np.eye(NH, dtype=np.float32),
                 np.ones((LP, LP), np.float32))
    cp = np.zeros(32, np.int32)
    cp[0::2] = np.arange(16)
    cp[1::2] = np.arange(16) + 16
    colperm = (np.arange(NH)[:, None] * 32 + cp[None, :]).reshape(-1)
    return perm, cst, sel, bd, colperm


_PERM, _CST, _SEL, _BD, _COLPERM = _host_constants()


def kernel(query, reference_points, value, value_spatial_shapes,
           W_off, b_off, W_attn, b_attn, W_val, b_val, W_out, b_out):
    wq = jnp.concatenate([W_off[_PERM], W_attn], axis=0).T
    bq = jnp.concatenate([b_off[_PERM], b_attn], axis=0).reshape(1, 384)

    vrows = _matmul(value.reshape(BS * NV, C), W_val.T[:, _COLPERM],
                    b_val[_COLPERM], 1360, out_dtype=jnp.bfloat16)
    vrows = vrows.reshape(NROWS, HD)

    idx, w = _qside(query.reshape(NR, C),
                    reference_points.reshape(NR, 2 * NL),
                    wq, bq,
                    jnp.asarray(_SEL), jnp.asarray(_BD), jnp.asarray(_CST))

    s = _sc_gather_fn()(idx, w, vrows)

    out = _matmul(s.reshape(NR, C), W_out.T, b_out, 720)
    return out.reshape(BS, NQ, C)


# speedup vs baseline: 1.3557x; 1.0266x over previous
import functools
import math

import numpy as np
import jax
import jax.numpy as jnp
from jax import lax
from jax.experimental import pallas as pl
from jax.experimental.pallas import tpu as pltpu
from jax.experimental.pallas import tpu_sc as plsc

SS = ((64, 64), (32, 32), (16, 16), (8, 8))
BS, NQ, C, NH, NL, NP = 4, 900, 256, 8, 4, 4
HD = C // NH
LP = NL * NP
NV = sum(h * w for h, w in SS)
NR = BS * NQ
NROWS = BS * NV * NH

RB = 16
NBLK = NR // RB
NTILES = 32
SC_ITERS = -(-NBLK // NTILES)


def _matmul_body(x_ref, w_ref, b_ref, o_ref):
    r = jnp.dot(x_ref[:], w_ref[:],
                preferred_element_type=jnp.float32,
                precision=lax.Precision.HIGHEST) + b_ref[:]
    o_ref[:] = r.astype(o_ref.dtype)


def _matmul(x, w, b, bm, out_dtype=jnp.float32):
    m, k = x.shape
    n = w.shape[1]
    return pl.pallas_call(
        _matmul_body,
        grid=(m // bm,),
        in_specs=[pl.BlockSpec((bm, k), lambda i: (i, 0)),
                  pl.BlockSpec((k, n), lambda i: (0, 0)),
                  pl.BlockSpec((1, n), lambda i: (0, 0))],
        out_specs=pl.BlockSpec((bm, n), lambda i: (i, 0)),
        out_shape=jax.ShapeDtypeStruct((m, n), out_dtype),
    )(x, w, b.reshape(1, n))


QBM = 400


def _qside_body(q_ref, rp_ref, wq_ref, bq_ref, sel_ref, bd_ref, cst_ref,
                idx_ref, w_ref):
    pid = pl.program_id(0)
    t = jnp.dot(q_ref[:], wq_ref[:],
                preferred_element_type=jnp.float32,
                precision=lax.Precision.HIGHEST) + bq_ref[:]
    offx = t[:, 0:128]
    offy = t[:, 128:256]
    logits = t[:, 256:384]

    m = jnp.max(logits, axis=1, keepdims=True)
    e = jnp.exp(logits - m)
    denom = jnp.dot(e, bd_ref[:], preferred_element_type=jnp.float32,
                    precision=lax.Precision.HIGHEST)
    aw = e / denom

    rxy = jnp.dot(rp_ref[:], sel_ref[:], preferred_element_type=jnp.float32,
                  precision=lax.Precision.HIGHEST)
    refx = rxy[:, 0:128]
    refy = rxy[:, 128:256]

    wl = cst_ref[0:1, :]
    hl = cst_ref[1:2, :]
    bl = cst_ref[2:3, :]
    hlane = cst_ref[3:4, :]

    ix = refx * wl + offx - 0.5
    iy = refy * hl + offy - 0.5
    big = 16777216.0
    ix = jnp.clip(ix, -4.0, big)
    iy = jnp.clip(iy, -4.0, big)
    x0 = jnp.floor(ix)
    fx = ix - x0
    y0 = jnp.floor(iy)
    fy = iy - y0

    rows = lax.broadcasted_iota(jnp.int32, (QBM, 1), 0) + pid * QBM
    bvec = (rows // NQ).astype(jnp.float32)
    base_all = bvec * float(NV) + bl

    for c, (cy, cx) in enumerate(((0, 0), (0, 1), (1, 0), (1, 1))):
        xq = x0 + float(cx)
        yq = y0 + float(cy)
        valid = ((xq >= 0.0) & (xq <= wl - 1.0)
                 & (yq >= 0.0) & (yq <= hl - 1.0)).astype(jnp.float32)
        xc = jnp.clip(xq, 0.0, wl - 1.0)
        yc = jnp.clip(yq, 0.0, hl - 1.0)
        flat = (base_all + yc * wl + xc) * float(NH) + hlane
        wx = (1.0 - fx) if cx == 0 else fx
        wy = (1.0 - fy) if cy == 0 else fy
        idx_ref[c, :, :] = flat.astype(jnp.int32)
        w_ref[c, :, :] = aw * wx * wy * valid


def _qside(q2, rp2, wq, bq, sel, bd, cst):
    return pl.pallas_call(
        _qside_body,
        grid=(NR // QBM,),
        in_specs=[pl.BlockSpec((QBM, C), lambda i: (i, 0)),
                  pl.BlockSpec((QBM, 2 * NL), lambda i: (i, 0)),
                  pl.BlockSpec((C, 384), lambda i: (0, 0)),
                  pl.BlockSpec((1, 384), lambda i: (0, 0)),
                  pl.BlockSpec((2 * NL, 256), lambda i: (0, 0)),
                  pl.BlockSpec((128, 128), lambda i: (0, 0)),
                  pl.BlockSpec((8, 128), lambda i: (0, 0))],
        out_specs=[pl.BlockSpec((4, QBM, 128), lambda i: (0, i, 0)),
                   pl.BlockSpec((4, QBM, 128), lambda i: (0, i, 0))],
        out_shape=[jax.ShapeDtypeStruct((4, NR, 128), jnp.int32),
                   jax.ShapeDtypeStruct((4, NR, 128), jnp.float32)],
    )(q2, rp2, wq, bq, sel, bd, cst)


def _sc_body(idx_hbm, w_hbm, vrows_hbm, out_hbm,
             idxv, wv, rows, outb, sem0, sem1, sem2, sem3, ssem0, ssem1):
    wid = lax.axis_index("s") * 2 + lax.axis_index("c")
    sems = (sem0, sem1, sem2, sem3)
    ssems = (ssem0, ssem1)

    def stage_fire(i, sb):
        bid = i * NTILES + wid

        @pl.when(bid < NBLK)
        def _():
            r0 = bid * RB
            pltpu.async_copy(idx_hbm.at[:, pl.ds(r0, RB)], idxv.at[sb],
                             ssems[sb])
            pltpu.async_copy(w_hbm.at[:, pl.ds(r0, RB)], wv.at[sb],
                             ssems[sb])

    def stage_wait(i, sb):
        bid = i * NTILES + wid

        @pl.when(bid < NBLK)
        def _():
            r0 = bid * RB
            pltpu.make_async_copy(idx_hbm.at[:, pl.ds(r0, RB)], idxv.at[sb],
                                  ssems[sb]).wait()
            pltpu.make_async_copy(w_hbm.at[:, pl.ds(r0, RB)], wv.at[sb],
                                  ssems[sb]).wait()

    def fire(j, sb, slot, sem):
        for c in range(4):
            pltpu.async_copy(vrows_hbm.at[idxv.at[sb, c, j]],
                             rows.at[slot, pl.ds(c * 128, 128)], sem)

    def drain(j, sb, slot, sem):
        for c in range(4):
            pltpu.make_async_copy(vrows_hbm.at[idxv.at[sb, c, j]],
                                  rows.at[slot, pl.ds(c * 128, 128)],
                                  sem).wait()

    def compute(j, sb, slot):
        def h_body(h, carry):
            hbase = h * 16
            parts0 = []
            parts1 = []
            hi_mask = jnp.full((16,), 0xFFFF0000, jnp.uint32)
            for c in range(4):
                w16 = wv[sb, c, j, pl.ds(hbase, 16)]
                a0e = jnp.zeros((16,), jnp.float32)
                a0o = jnp.zeros((16,), jnp.float32)
                a1e = jnp.zeros((16,), jnp.float32)
                a1o = jnp.zeros((16,), jnp.float32)
                for lp in range(16):
                    wb = jnp.full((16,), w16[lp], jnp.float32)
                    ri = hbase + (c * 128 + lp)
                    u = plsc.bitcast(rows[slot, ri, :], jnp.uint32)
                    r0 = plsc.bitcast(u << 16, jnp.float32)
                    r1 = plsc.bitcast(u & hi_mask, jnp.float32)
                    if lp % 2 == 0:
                        a0e = a0e + wb * r0
                        a1e = a1e + wb * r1
                    else:
                        a0o = a0o + wb * r0
                        a1o = a1o + wb * r1
                parts0.append(a0e + a0o)
                parts1.append(a1e + a1o)
            acc0 = (parts0[0] + parts0[1]) + (parts0[2] + parts0[3])
            acc1 = (parts1[0] + parts1[1]) + (parts1[2] + parts1[3])
            outb[j * 8 + h, pl.ds(0, 16)] = acc0
            outb[j * 8 + h, pl.ds(16, 16)] = acc1
            return carry
        lax.fori_loop(0, NH, h_body, 0)

    def process(i, sb):
        bid = i * NTILES + wid

        @pl.when(bid < NBLK)
        def _():
            r0 = bid * RB
            fire(0, sb, 0, sems[0])
            fire(1, sb, 1, sems[1])
            fire(2, sb, 2, sems[2])

            def quad_body(p, c2):
                j = 4 * p
                for q in range(4):
                    jq = j + q

                    @pl.when(jq + 3 < RB)
                    def _():
                        fire(jq + 3, sb, (q + 3) % 4, sems[(q + 3) % 4])
                    drain(jq, sb, q, sems[q])
                    compute(jq, sb, q)
                return c2
            lax.fori_loop(0, RB // 4, quad_body, 0)
            pltpu.sync_copy(outb, out_hbm.at[pl.ds(r0 * NH, RB * NH)])

    stage_fire(0, 0)

    def blk_body(t, carry):
        for k, sb in ((0, 0), (1, 1)):
            i = 2 * t + k
            stage_fire(i + 1, 1 - sb)
            stage_wait(i, sb)
            process(i, sb)
        return carry

    lax.fori_loop(0, SC_ITERS // 2, blk_body, 0)


@functools.cache
def _sc_gather_fn():
    mesh = plsc.VectorSubcoreMesh(core_axis_name="c", subcore_axis_name="s",
                                  num_cores=2, num_subcores=16)
    return pl.kernel(
        _sc_body,
        out_type=jax.ShapeDtypeStruct((NR * NH, HD), jnp.float32),
        mesh=mesh,
        scratch_types=[
            pltpu.VMEM((2, 4, RB, 128), jnp.int32),
            pltpu.VMEM((2, 4, RB, 128), jnp.float32),
            pltpu.VMEM((4, 512, HD), jnp.bfloat16),
            pltpu.VMEM((RB * NH, HD), jnp.float32),
            pltpu.SemaphoreType.DMA,
            pltpu.SemaphoreType.DMA,
            pltpu.SemaphoreType.DMA,
            pltpu.SemaphoreType.DMA,
            pltpu.SemaphoreType.DMA,
            pltpu.SemaphoreType.DMA,
        ],
        compiler_params=pltpu.CompilerParams(use_tc_tiling_on_sc=False,
                                             needs_layout_passes=False),
    )


def _host_constants():
    hlp = np.arange(NH * NL * NP)
    perm = np.concatenate([hlp * 2 + 0, hlp * 2 + 1]).astype(np.int32)
    l_of_lane = (hlp % LP) // NP
    h_of_lane = hlp // LP
    wl = np.array([SS[l][1] for l in l_of_lane], np.float32)
    hl = np.array([SS[l][0] for l in l_of_lane], np.float32)
    bases = np.cumsum([0] + [h * w for h, w in SS])[:-1]
    bl = np.array([bases[l] for l in l_of_lane], np.float32)
    cst = np.zeros((8, 128), np.float32)
    cst[0] = wl
    cst[1] = hl
    cst[2] = bl
    cst[3] = h_of_lane.astype(np.float32)
    sel = np.zeros((2 * NL, 256), np.float32)
    for lane in range(128):
        sel[l_of_lane[lane] * 2 + 0, lane] = 1.0
        sel[l_of_lane[lane] * 2 + 1, 128 + lane] = 1.0
    bd = np.kron(np.eye(NH, dtype=np.float32),
                 np.ones((LP, LP), np.float32))
    cp = np.zeros(32, np.int32)
    cp[0::2] = np.arange(16)
    cp[1::2] = np.arange(16) + 16
    colperm = (np.arange(NH)[:, None] * 32 + cp[None, :]).reshape(-1)
    return perm, cst, sel, bd, colperm


_PERM, _CST, _SEL, _BD, _COLPERM = _host_constants()


def kernel(query, reference_points, value, value_spatial_shapes,
           W_off, b_off, W_attn, b_attn, W_val, b_val, W_out, b_out):
    wq = jnp.concatenate([W_off[_PERM], W_attn], axis=0).T
    bq = jnp.concatenate([b_off[_PERM], b_attn], axis=0).reshape(1, 384)

    vrows = _matmul(value.reshape(BS * NV, C), W_val.T[:, _COLPERM],
                    b_val[_COLPERM], 1360, out_dtype=jnp.bfloat16)
    vrows = vrows.reshape(NROWS, HD)

    idx, w = _qside(query.reshape(NR, C),
                    reference_points.reshape(NR, 2 * NL),
                    wq, bq,
                    jnp.asarray(_SEL), jnp.asarray(_BD), jnp.asarray(_CST))

    s = _sc_gather_fn()(idx, w, vrows)

    out = _matmul(s.reshape(NR, C), W_out.T, b_out, 720)
    return out.reshape(BS, NQ, C)


# speedup vs baseline: 1.4016x; 1.0338x over previous
import functools
import math

import numpy as np
import jax
import jax.numpy as jnp
from jax import lax
from jax.experimental import pallas as pl
from jax.experimental.pallas import tpu as pltpu
from jax.experimental.pallas import tpu_sc as plsc

SS = ((64, 64), (32, 32), (16, 16), (8, 8))
BS, NQ, C, NH, NL, NP = 4, 900, 256, 8, 4, 4
HD = C // NH
LP = NL * NP
NV = sum(h * w for h, w in SS)
NR = BS * NQ
NROWS = BS * NV * NH

RB = 16
NBLK = NR // RB
NTILES = 32
SC_ITERS = -(-NBLK // NTILES)


def _matmul_body(x_ref, w_ref, b_ref, o_ref):
    r = jnp.dot(x_ref[:], w_ref[:],
                preferred_element_type=jnp.float32) + b_ref[:]
    o_ref[:] = r.astype(o_ref.dtype)


def _matmul(x, w, b, bm, out_dtype=jnp.float32):
    m, k = x.shape
    n = w.shape[1]
    return pl.pallas_call(
        _matmul_body,
        grid=(m // bm,),
        in_specs=[pl.BlockSpec((bm, k), lambda i: (i, 0)),
                  pl.BlockSpec((k, n), lambda i: (0, 0)),
                  pl.BlockSpec((1, n), lambda i: (0, 0))],
        out_specs=pl.BlockSpec((bm, n), lambda i: (i, 0)),
        out_shape=jax.ShapeDtypeStruct((m, n), out_dtype),
    )(x, w, b.reshape(1, n))


QBM = 400


def _qside_body(q_ref, rp_ref, wq_ref, bq_ref, sel_ref, bd_ref, cst_ref,
                idx_ref, w_ref):
    pid = pl.program_id(0)
    t = jnp.dot(q_ref[:], wq_ref[:],
                preferred_element_type=jnp.float32,
                precision=lax.Precision.HIGHEST) + bq_ref[:]
    offx = t[:, 0:128]
    offy = t[:, 128:256]
    logits = t[:, 256:384]

    m = jnp.max(logits, axis=1, keepdims=True)
    e = jnp.exp(logits - m)
    denom = jnp.dot(e, bd_ref[:], preferred_element_type=jnp.float32,
                    precision=lax.Precision.HIGHEST)
    aw = e / denom

    rxy = jnp.dot(rp_ref[:], sel_ref[:], preferred_element_type=jnp.float32,
                  precision=lax.Precision.HIGHEST)
    refx = rxy[:, 0:128]
    refy = rxy[:, 128:256]

    wl = cst_ref[0:1, :]
    hl = cst_ref[1:2, :]
    bl = cst_ref[2:3, :]
    hlane = cst_ref[3:4, :]

    ix = refx * wl + offx - 0.5
    iy = refy * hl + offy - 0.5
    big = 16777216.0
    ix = jnp.clip(ix, -4.0, big)
    iy = jnp.clip(iy, -4.0, big)
    x0 = jnp.floor(ix)
    fx = ix - x0
    y0 = jnp.floor(iy)
    fy = iy - y0

    rows = lax.broadcasted_iota(jnp.int32, (QBM, 1), 0) + pid * QBM
    bvec = (rows // NQ).astype(jnp.float32)
    base_all = bvec * float(NV) + bl

    for c, (cy, cx) in enumerate(((0, 0), (0, 1), (1, 0), (1, 1))):
        xq = x0 + float(cx)
        yq = y0 + float(cy)
        valid = ((xq >= 0.0) & (xq <= wl - 1.0)
                 & (yq >= 0.0) & (yq <= hl - 1.0)).astype(jnp.float32)
        xc = jnp.clip(xq, 0.0, wl - 1.0)
        yc = jnp.clip(yq, 0.0, hl - 1.0)
        flat = (base_all + yc * wl + xc) * float(NH) + hlane
        wx = (1.0 - fx) if cx == 0 else fx
        wy = (1.0 - fy) if cy == 0 else fy
        idx_ref[c, :, :] = flat.astype(jnp.int32)
        w_ref[c, :, :] = aw * wx * wy * valid


def _qside(q2, rp2, wq, bq, sel, bd, cst):
    return pl.pallas_call(
        _qside_body,
        grid=(NR // QBM,),
        in_specs=[pl.BlockSpec((QBM, C), lambda i: (i, 0)),
                  pl.BlockSpec((QBM, 2 * NL), lambda i: (i, 0)),
                  pl.BlockSpec((C, 384), lambda i: (0, 0)),
                  pl.BlockSpec((1, 384), lambda i: (0, 0)),
                  pl.BlockSpec((2 * NL, 256), lambda i: (0, 0)),
                  pl.BlockSpec((128, 128), lambda i: (0, 0)),
                  pl.BlockSpec((8, 128), lambda i: (0, 0))],
        out_specs=[pl.BlockSpec((4, QBM, 128), lambda i: (0, i, 0)),
                   pl.BlockSpec((4, QBM, 128), lambda i: (0, i, 0))],
        out_shape=[jax.ShapeDtypeStruct((4, NR, 128), jnp.int32),
                   jax.ShapeDtypeStruct((4, NR, 128), jnp.float32)],
    )(q2, rp2, wq, bq, sel, bd, cst)


def _sc_body(idx_hbm, w_hbm, vrows_hbm, out_hbm,
             idxv, wv, rows, outb, sem0, sem1, sem2, sem3, ssem0, ssem1):
    wid = lax.axis_index("s") * 2 + lax.axis_index("c")
    sems = (sem0, sem1, sem2, sem3)
    ssems = (ssem0, ssem1)

    def stage_fire(i, sb):
        bid = i * NTILES + wid

        @pl.when(bid < NBLK)
        def _():
            r0 = bid * RB
            pltpu.async_copy(idx_hbm.at[:, pl.ds(r0, RB)], idxv.at[sb],
                             ssems[sb])
            pltpu.async_copy(w_hbm.at[:, pl.ds(r0, RB)], wv.at[sb],
                             ssems[sb])

    def stage_wait(i, sb):
        bid = i * NTILES + wid

        @pl.when(bid < NBLK)
        def _():
            r0 = bid * RB
            pltpu.make_async_copy(idx_hbm.at[:, pl.ds(r0, RB)], idxv.at[sb],
                                  ssems[sb]).wait()
            pltpu.make_async_copy(w_hbm.at[:, pl.ds(r0, RB)], wv.at[sb],
                                  ssems[sb]).wait()

    def fire(j, sb, slot, sem):
        for c in range(4):
            pltpu.async_copy(vrows_hbm.at[idxv.at[sb, c, j]],
                             rows.at[slot, pl.ds(c * 128, 128)], sem)

    def drain(j, sb, slot, sem):
        for c in range(4):
            pltpu.make_async_copy(vrows_hbm.at[idxv.at[sb, c, j]],
                                  rows.at[slot, pl.ds(c * 128, 128)],
                                  sem).wait()

    def compute(j, sb, slot):
        def h_body(h, carry):
            hbase = h * 16
            parts0 = []
            parts1 = []
            hi_mask = jnp.full((16,), 0xFFFF0000, jnp.uint32)
            for c in range(4):
                w16 = wv[sb, c, j, pl.ds(hbase, 16)]
                a0e = jnp.zeros((16,), jnp.float32)
                a0o = jnp.zeros((16,), jnp.float32)
                a1e = jnp.zeros((16,), jnp.float32)
                a1o = jnp.zeros((16,), jnp.float32)
                for lp in range(16):
                    wb = jnp.full((16,), w16[lp], jnp.float32)
                    ri = hbase + (c * 128 + lp)
                    u = plsc.bitcast(rows[slot, ri, :], jnp.uint32)
                    r0 = plsc.bitcast(u << 16, jnp.float32)
                    r1 = plsc.bitcast(u & hi_mask, jnp.float32)
                    if lp % 2 == 0:
                        a0e = a0e + wb * r0
                        a1e = a1e + wb * r1
                    else:
                        a0o = a0o + wb * r0
                        a1o = a1o + wb * r1
                parts0.append(a0e + a0o)
                parts1.append(a1e + a1o)
            acc0 = (parts0[0] + parts0[1]) + (parts0[2] + parts0[3])
            acc1 = (parts1[0] + parts1[1]) + (parts1[2] + parts1[3])
            outb[j * 8 + h, pl.ds(0, 16)] = acc0
            outb[j * 8 + h, pl.ds(16, 16)] = acc1
            return carry
        lax.fori_loop(0, NH, h_body, 0)

    def process(i, sb):
        bid = i * NTILES + wid

        @pl.when(bid < NBLK)
        def _():
            r0 = bid * RB
            fire(0, sb, 0, sems[0])
            fire(1, sb, 1, sems[1])
            fire(2, sb, 2, sems[2])

            def quad_body(p, c2):
                j = 4 * p
                for q in range(4):
                    jq = j + q

                    @pl.when(jq + 3 < RB)
                    def _():
                        fire(jq + 3, sb, (q + 3) % 4, sems[(q + 3) % 4])
                    drain(jq, sb, q, sems[q])
                    compute(jq, sb, q)
                return c2
            lax.fori_loop(0, RB // 4, quad_body, 0)
            pltpu.sync_copy(outb, out_hbm.at[pl.ds(r0 * NH, RB * NH)])

    stage_fire(0, 0)

    def blk_body(t, carry):
        for k, sb in ((0, 0), (1, 1)):
            i = 2 * t + k
            stage_fire(i + 1, 1 - sb)
            stage_wait(i, sb)
            process(i, sb)
        return carry

    lax.fori_loop(0, SC_ITERS // 2, blk_body, 0)


@functools.cache
def _sc_gather_fn():
    mesh = plsc.VectorSubcoreMesh(core_axis_name="c", subcore_axis_name="s",
                                  num_cores=2, num_subcores=16)
    return pl.kernel(
        _sc_body,
        out_type=jax.ShapeDtypeStruct((NR * NH, HD), jnp.float32),
        mesh=mesh,
        scratch_types=[
            pltpu.VMEM((2, 4, RB, 128), jnp.int32),
            pltpu.VMEM((2, 4, RB, 128), jnp.float32),
            pltpu.VMEM((4, 512, HD), jnp.bfloat16),
            pltpu.VMEM((RB * NH, HD), jnp.float32),
            pltpu.SemaphoreType.DMA,
            pltpu.SemaphoreType.DMA,
            pltpu.SemaphoreType.DMA,
            pltpu.SemaphoreType.DMA,
            pltpu.SemaphoreType.DMA,
            pltpu.SemaphoreType.DMA,
        ],
        compiler_params=pltpu.CompilerParams(use_tc_tiling_on_sc=False,
                                             needs_layout_passes=False),
    )


def _host_constants():
    hlp = np.arange(NH * NL * NP)
    perm = np.concatenate([hlp * 2 + 0, hlp * 2 + 1]).astype(np.int32)
    l_of_lane = (hlp % LP) // NP
    h_of_lane = hlp // LP
    wl = np.array([SS[l][1] for l in l_of_lane], np.float32)
    hl = np.array([SS[l][0] for l in l_of_lane], np.float32)
    bases = np.cumsum([0] + [h * w for h, w in SS])[:-1]
    bl = np.array([bases[l] for l in l_of_lane], np.float32)
    cst = np.zeros((8, 128), np.float32)
    cst[0] = wl
    cst[1] = hl
    cst[2] = bl
    cst[3] = h_of_lane.astype(np.float32)
    sel = np.zeros((2 * NL, 256), np.float32)
    for lane in range(128):
        sel[l_of_lane[lane] * 2 + 0, lane] = 1.0
        sel[l_of_lane[lane] * 2 + 1, 128 + lane] = 1.0
    bd = np.kron(np.eye(NH, dtype=np.float32),
                 np.ones((LP, LP), np.float32))
    cp = np.zeros(32, np.int32)
    cp[0::2] = np.arange(16)
    cp[1::2] = np.arange(16) + 16
    colperm = (np.arange(NH)[:, None] * 32 + cp[None, :]).reshape(-1)
    return perm, cst, sel, bd, colperm


_PERM, _CST, _SEL, _BD, _COLPERM = _host_constants()


def kernel(query, reference_points, value, value_spatial_shapes,
           W_off, b_off, W_attn, b_attn, W_val, b_val, W_out, b_out):
    wq = jnp.concatenate([W_off[_PERM], W_attn], axis=0).T
    bq = jnp.concatenate([b_off[_PERM], b_attn], axis=0).reshape(1, 384)

    vrows = _matmul(value.reshape(BS * NV, C), W_val.T[:, _COLPERM],
                    b_val[_COLPERM], 1360, out_dtype=jnp.bfloat16)
    vrows = vrows.reshape(NROWS, HD)

    idx, w = _qside(query.reshape(NR, C),
                    reference_points.reshape(NR, 2 * NL),
                    wq, bq,
                    jnp.asarray(_SEL), jnp.asarray(_BD), jnp.asarray(_CST))

    s = _sc_gather_fn()(idx, w, vrows)

    out = _matmul(s.reshape(NR, C), W_out.T, b_out, 720)
    return out.reshape(BS, NQ, C)
